# baseline (speedup 1.0000x reference)
import jax
import jax.numpy as jnp
from jax import lax
from jax.experimental import pallas as pl
from jax.experimental.pallas import tpu as pltpu
from jax.experimental.pallas import tpu_sc as plsc

N_NODES = 10000
N_EDGES = 320000
D_FEAT = 128
HIDDEN = 128
N_CLASSES = 2

NC = 2
NS = 16
NW = NC * NS

CK = 128
CHUNKS = 80
EPT = CHUNKS * CK
E_PAD = NW * EPT
L = 16

N_PAD = 10240
ROWS_PER_TILE = N_PAD // NS


BT = 1024


def _tc_body(agg_ref, cnt_ref, w1_ref, b1_ref, w2_ref, b2_ref, out_ref):
  agg = agg_ref[0] + agg_ref[1]
  deg = jnp.sum(jnp.transpose(cnt_ref[...]), axis=1, keepdims=True)
  agg = agg / jnp.maximum(deg, 1.0)
  h = jnp.dot(agg, w1_ref[...], preferred_element_type=jnp.float32)
  h = jnp.maximum(h + b1_ref[...], 0.0)
  logits = jnp.dot(h, w2_ref[...], preferred_element_type=jnp.float32)
  logits = logits + b2_ref[...]
  m = jnp.max(logits, axis=1, keepdims=True)
  lse = m + jnp.log(jnp.sum(jnp.exp(logits - m), axis=1, keepdims=True))
  out_ref[...] = logits - lse


def _sc_body(edge_hbm, x_hbm, zagg_hbm,
             agg_out, cnt_out,
             edge_all, rows_v, cnt_v, agg_sh):
  cid = lax.axis_index("c")
  sid = lax.axis_index("s")
  wid = cid * NS + sid

  row0 = sid * ROWS_PER_TILE
  pltpu.sync_copy(zagg_hbm.at[pl.ds(row0, ROWS_PER_TILE)],
                  agg_sh.at[pl.ds(row0, ROWS_PER_TILE)])

  zvec = jnp.zeros((L,), jnp.float32)

  @pl.loop(0, N_PAD // L)
  def _zero_cnt(i):
    cnt_v[pl.ds(i * L, L)] = zvec

  pltpu.sync_copy(edge_hbm.at[wid], edge_all)
  plsc.subcore_barrier()

  ones16 = jnp.ones((L,), jnp.float32)

  @pl.loop(0, CHUNKS)
  def _edge_chunk(c):
    pltpu.sync_copy(x_hbm.at[edge_all.at[c, 0]], rows_v)
    pltpu.sync_copy(rows_v, agg_sh.at[edge_all.at[c, 1]], add=True)
    for j in range(CK // L):
      idx = edge_all[c, 1, pl.ds(j * L, L)]
      plsc.addupdate_scatter(cnt_v, [idx], ones16)

  plsc.subcore_barrier()

  pltpu.sync_copy(agg_sh.at[pl.ds(row0, ROWS_PER_TILE)],
                  agg_out.at[cid, pl.ds(row0, ROWS_PER_TILE)])
  pltpu.sync_copy(cnt_v, cnt_out.at[wid])


@jax.jit
def kernel(x, edge_index, W1, b1, W2, b2):
  edges = edge_index.astype(jnp.int32)
  pad = jnp.concatenate(
      [jnp.zeros((1, E_PAD - N_EDGES), jnp.int32),
       jnp.full((1, E_PAD - N_EDGES), N_NODES, jnp.int32)], 0)
  edges = jnp.concatenate([edges, pad], 1)
  edges = (edges.reshape(2, NW, EPT).transpose(1, 0, 2)
           .reshape(NW, 2, CHUNKS, CK).transpose(0, 2, 1, 3))

  zagg = jnp.zeros((N_PAD, D_FEAT), jnp.float32)

  mesh = plsc.VectorSubcoreMesh(core_axis_name="c", subcore_axis_name="s")
  agg_parts, cnt_parts = pl.kernel(
      _sc_body,
      out_type=(
          jax.ShapeDtypeStruct((NC, N_PAD, D_FEAT), jnp.float32),
          jax.ShapeDtypeStruct((NW, N_PAD), jnp.float32),
      ),
      mesh=mesh,
      compiler_params=pltpu.CompilerParams(needs_layout_passes=False),
      scratch_types=[
          pltpu.VMEM((CHUNKS, 2, CK), jnp.int32),
          pltpu.VMEM((CK, D_FEAT), jnp.float32),
          pltpu.VMEM((N_PAD,), jnp.float32),
          pltpu.VMEM_SHARED((N_PAD, D_FEAT), jnp.float32),
      ],
  )(edges, x, zagg)

  out = pl.pallas_call(
      _tc_body,
      grid=(pl.cdiv(N_NODES, BT),),
      in_specs=[
          pl.BlockSpec((NC, BT, D_FEAT), lambda i: (0, i, 0)),
          pl.BlockSpec((NW, BT), lambda i: (0, i)),
          pl.BlockSpec((D_FEAT, HIDDEN), lambda i: (0, 0)),
          pl.BlockSpec((1, HIDDEN), lambda i: (0, 0)),
          pl.BlockSpec((HIDDEN, N_CLASSES), lambda i: (0, 0)),
          pl.BlockSpec((1, N_CLASSES), lambda i: (0, 0)),
      ],
      out_specs=pl.BlockSpec((BT, N_CLASSES), lambda i: (i, 0)),
      out_shape=jax.ShapeDtypeStruct((N_NODES, N_CLASSES), jnp.float32),
  )(agg_parts, cnt_parts, W1, b1.reshape(1, HIDDEN),
    W2, b2.reshape(1, N_CLASSES))
  return out


# speedup vs baseline: 1.5094x; 1.5094x over previous
import jax
import jax.numpy as jnp
from jax import lax
from jax.experimental import pallas as pl
from jax.experimental.pallas import tpu as pltpu
from jax.experimental.pallas import tpu_sc as plsc

N_NODES = 10000
N_EDGES = 320000
D_FEAT = 128
HIDDEN = 128
N_CLASSES = 2

NC = 2
NS = 16
NW = NC * NS

CK = 128
CHUNKS = 79
E_PAD = CHUNKS * NW * CK
L = 16

N_PAD = 10240
ROWS_PER_TILE = N_PAD // NS


BT = 1024


def _tc_body(agg_ref, cnt_ref, w1_ref, b1_ref, w2_ref, b2_ref, out_ref):
  agg = agg_ref[0] + agg_ref[1]
  deg = jnp.sum(jnp.transpose(cnt_ref[...]), axis=1, keepdims=True)
  agg = agg / jnp.maximum(deg, 1.0)
  h = jnp.dot(agg, w1_ref[...], preferred_element_type=jnp.float32)
  h = jnp.maximum(h + b1_ref[...], 0.0)
  logits = jnp.dot(h, w2_ref[...], preferred_element_type=jnp.float32)
  logits = logits + b2_ref[...]
  m = jnp.max(logits, axis=1, keepdims=True)
  lse = m + jnp.log(jnp.sum(jnp.exp(logits - m), axis=1, keepdims=True))
  out_ref[...] = logits - lse


def _sc_body(edge_hbm, x_hbm, zagg_hbm,
             agg_out, cnt_out,
             edge_v, rows_v, cnt_v, agg_sh):
  cid = lax.axis_index("c")
  sid = lax.axis_index("s")
  wid = cid * NS + sid

  row0 = sid * ROWS_PER_TILE
  pltpu.sync_copy(zagg_hbm.at[pl.ds(row0, ROWS_PER_TILE)],
                  agg_sh.at[pl.ds(row0, ROWS_PER_TILE)])

  zvec = jnp.zeros((L,), jnp.float32)

  @pl.loop(0, N_PAD // L)
  def _zero_cnt(i):
    cnt_v[pl.ds(i * L, L)] = zvec

  plsc.subcore_barrier()

  ones16 = jnp.ones((L,), jnp.float32)

  @plsc.parallel_loop(0, CHUNKS, unroll=2)
  def _edge_chunk(c):
    b = lax.rem(c, 2)
    off = (c * NW + wid) * CK
    pltpu.sync_copy(edge_hbm.at[:, pl.ds(off, CK)], edge_v.at[b])
    pltpu.sync_copy(x_hbm.at[edge_v.at[b, 0]], rows_v.at[b])
    pltpu.sync_copy(rows_v.at[b], agg_sh.at[edge_v.at[b, 1]], add=True)
    for j in range(CK // L):
      idx = edge_v[b, 1, pl.ds(j * L, L)]
      plsc.addupdate_scatter(cnt_v, [idx], ones16)

  plsc.subcore_barrier()

  pltpu.sync_copy(agg_sh.at[pl.ds(row0, ROWS_PER_TILE)],
                  agg_out.at[cid, pl.ds(row0, ROWS_PER_TILE)])
  pltpu.sync_copy(cnt_v, cnt_out.at[wid])


@jax.jit
def kernel(x, edge_index, W1, b1, W2, b2):
  edges = edge_index.astype(jnp.int32)
  pad = jnp.concatenate(
      [jnp.zeros((1, E_PAD - N_EDGES), jnp.int32),
       jnp.full((1, E_PAD - N_EDGES), N_NODES, jnp.int32)], 0)
  edges = jnp.concatenate([edges, pad], 1)

  zagg = jnp.zeros((N_PAD, D_FEAT), jnp.float32)

  mesh = plsc.VectorSubcoreMesh(core_axis_name="c", subcore_axis_name="s")
  agg_parts, cnt_parts = pl.kernel(
      _sc_body,
      out_type=(
          jax.ShapeDtypeStruct((NC, N_PAD, D_FEAT), jnp.float32),
          jax.ShapeDtypeStruct((NW, N_PAD), jnp.float32),
      ),
      mesh=mesh,
      compiler_params=pltpu.CompilerParams(needs_layout_passes=False),
      scratch_types=[
          pltpu.VMEM((2, 2, CK), jnp.int32),
          pltpu.VMEM((2, CK, D_FEAT), jnp.float32),
          pltpu.VMEM((N_PAD,), jnp.float32),
          pltpu.VMEM_SHARED((N_PAD, D_FEAT), jnp.float32),
      ],
  )(edges, x, zagg)

  out = pl.pallas_call(
      _tc_body,
      grid=(pl.cdiv(N_NODES, BT),),
      in_specs=[
          pl.BlockSpec((NC, BT, D_FEAT), lambda i: (0, i, 0)),
          pl.BlockSpec((NW, BT), lambda i: (0, i)),
          pl.BlockSpec((D_FEAT, HIDDEN), lambda i: (0, 0)),
          pl.BlockSpec((1, HIDDEN), lambda i: (0, 0)),
          pl.BlockSpec((HIDDEN, N_CLASSES), lambda i: (0, 0)),
          pl.BlockSpec((1, N_CLASSES), lambda i: (0, 0)),
      ],
      out_specs=pl.BlockSpec((BT, N_CLASSES), lambda i: (i, 0)),
      out_shape=jax.ShapeDtypeStruct((N_NODES, N_CLASSES), jnp.float32),
  )(agg_parts, cnt_parts, W1, b1.reshape(1, HIDDEN),
    W2, b2.reshape(1, N_CLASSES))
  return out
